# Initial kernel scaffold; baseline (speedup 1.0000x reference)
import jax
import jax.numpy as jnp
from jax.experimental import pallas as pl


def kernel(xy, grid):
    raise NotImplementedError("write your pallas kernel here")


# speedup vs baseline: 1.1092x; 1.1092x over previous
import functools

import jax
import jax.numpy as jnp
from jax import lax
from jax.experimental import pallas as pl
from jax.experimental.pallas import tpu as pltpu
from jax.experimental.pallas import tpu_sc as plsc

_NC = 2
_NS = 16
_L = 16
_NW = _NC * _NS


@functools.lru_cache(maxsize=None)
def _build(P, H, W, C):
    PW = P // _NW
    n_chunks = PW // C
    mesh = plsc.VectorSubcoreMesh(
        core_axis_name="c", subcore_axis_name="s",
        num_cores=_NC, num_subcores=_NS)

    @functools.partial(
        pl.kernel,
        out_type=jax.ShapeDtypeStruct((P,), jnp.float32),
        mesh=mesh,
        scratch_types=[
            pltpu.VMEM((C,), jnp.float32),
            pltpu.VMEM((C,), jnp.float32),
            pltpu.VMEM((4 * C,), jnp.int32),
            pltpu.VMEM((4 * C,), jnp.float32),
            pltpu.VMEM((C,), jnp.float32),
            pltpu.VMEM((C,), jnp.float32),
            pltpu.VMEM((C,), jnp.float32),
            pltpu.SemaphoreType.DMA,
        ],
    )
    def grid_sample(xy_hbm, g_hbm, out_hbm, xv, yv, idxv, valv, wxv, wyv,
                    outv, sem):
        wid = lax.axis_index("s") * _NC + lax.axis_index("c")
        base0 = wid * PW
        fw = jnp.float32(W - 1)
        fh = jnp.float32(H - 1)

        @pl.loop(0, n_chunks)
        def _chunk(k):
            base = base0 + k * C
            pltpu.sync_copy(xy_hbm.at[0, pl.ds(base, C)], xv)
            pltpu.sync_copy(xy_hbm.at[1, pl.ds(base, C)], yv)

            @pl.loop(0, C // _L)
            def _indices(j):
                o = j * _L
                xs = xv[pl.ds(o, _L)]
                ys = yv[pl.ds(o, _L)]
                xf = xs * fw
                yf = ys * fh
                x0 = jnp.clip(xf.astype(jnp.int32), 0, W - 1)
                y0 = jnp.clip(yf.astype(jnp.int32), 0, H - 1)
                wxv[pl.ds(o, _L)] = xf - x0.astype(jnp.float32)
                wyv[pl.ds(o, _L)] = yf - y0.astype(jnp.float32)
                x1 = jnp.minimum(x0 + 1, W - 1)
                r0 = y0 * W
                r1 = jnp.minimum(y0 + 1, H - 1) * W
                idxv[pl.ds(o, _L)] = r0 + x0
                idxv[pl.ds(C + o, _L)] = r0 + x1
                idxv[pl.ds(2 * C + o, _L)] = r1 + x0
                idxv[pl.ds(3 * C + o, _L)] = r1 + x1

            pltpu.async_copy(g_hbm.at[idxv], valv, sem).wait()

            @pl.loop(0, C // _L)
            def _blend(j):
                o = j * _L
                v00 = valv[pl.ds(o, _L)]
                v01 = valv[pl.ds(C + o, _L)]
                v10 = valv[pl.ds(2 * C + o, _L)]
                v11 = valv[pl.ds(3 * C + o, _L)]
                wx = wxv[pl.ds(o, _L)]
                wy = wyv[pl.ds(o, _L)]
                top = v00 + wx * (v01 - v00)
                bot = v10 + wx * (v11 - v10)
                outv[pl.ds(o, _L)] = top + wy * (bot - top)

            pltpu.sync_copy(outv, out_hbm.at[pl.ds(base, C)])

    return grid_sample


def kernel(xy, grid):
    P = xy.shape[0]
    H, W = grid.shape[-2], grid.shape[-1]
    return _build(P, H, W, 2048)(xy.T, grid.reshape(-1))


# speedup vs baseline: 1.4669x; 1.3225x over previous
import functools

import jax
import jax.numpy as jnp
from jax import lax
from jax.experimental import pallas as pl
from jax.experimental.pallas import tpu as pltpu
from jax.experimental.pallas import tpu_sc as plsc

_NC = 2
_NS = 16
_L = 16
_NW = _NC * _NS


@functools.lru_cache(maxsize=None)
def _build(P, H, W, C):
    PW = P // _NW
    n_chunks = PW // C
    mesh = plsc.VectorSubcoreMesh(
        core_axis_name="c", subcore_axis_name="s",
        num_cores=_NC, num_subcores=_NS)

    vmem_f = lambda n: pltpu.VMEM((n,), jnp.float32)
    vmem_i = lambda n: pltpu.VMEM((n,), jnp.int32)

    @functools.partial(
        pl.kernel,
        out_type=jax.ShapeDtypeStruct((P,), jnp.float32),
        mesh=mesh,
        scratch_types=[
            [vmem_f(C)] * 2,
            [vmem_f(C)] * 2,
            [vmem_i(4 * C)] * 2,
            [vmem_f(4 * C)] * 2,
            [vmem_f(C)] * 2,
            [vmem_f(C)] * 2,
            [vmem_f(C)] * 2,
            [pltpu.SemaphoreType.DMA] * 2,
            [pltpu.SemaphoreType.DMA] * 2,
        ],
    )
    def grid_sample(xy_hbm, g_hbm, out_hbm, xv, yv, idxv, valv, wxv, wyv,
                    outv, sx, sg):
        wid = lax.axis_index("s") * _NC + lax.axis_index("c")
        base0 = wid * PW
        fw = jnp.float32(W - 1)
        fh = jnp.float32(H - 1)

        def start_load(k, b):
            base = base0 + k * C
            a = pltpu.async_copy(xy_hbm.at[0, pl.ds(base, C)], xv[b], sx[b])
            c = pltpu.async_copy(xy_hbm.at[1, pl.ds(base, C)], yv[b], sx[b])
            return (a, c)

        def compute_idx(b):
            @pl.loop(0, C // _L)
            def _indices(j):
                o = j * _L
                xf = xv[b][pl.ds(o, _L)] * fw
                yf = yv[b][pl.ds(o, _L)] * fh
                x0 = jnp.clip(xf.astype(jnp.int32), 0, W - 1)
                y0 = jnp.clip(yf.astype(jnp.int32), 0, H - 1)
                wxv[b][pl.ds(o, _L)] = xf - x0.astype(jnp.float32)
                wyv[b][pl.ds(o, _L)] = yf - y0.astype(jnp.float32)
                x1 = jnp.minimum(x0 + 1, W - 1)
                r0 = y0 * W
                r1 = jnp.minimum(y0 + 1, H - 1) * W
                idxv[b][pl.ds(o, _L)] = r0 + x0
                idxv[b][pl.ds(C + o, _L)] = r0 + x1
                idxv[b][pl.ds(2 * C + o, _L)] = r1 + x0
                idxv[b][pl.ds(3 * C + o, _L)] = r1 + x1

        def start_gather(b):
            return pltpu.async_copy(g_hbm.at[idxv[b]], valv[b], sg[b])

        def blend(b):
            @pl.loop(0, C // _L)
            def _blend(j):
                o = j * _L
                v00 = valv[b][pl.ds(o, _L)]
                v01 = valv[b][pl.ds(C + o, _L)]
                v10 = valv[b][pl.ds(2 * C + o, _L)]
                v11 = valv[b][pl.ds(3 * C + o, _L)]
                wx = wxv[b][pl.ds(o, _L)]
                wy = wyv[b][pl.ds(o, _L)]
                top = v00 + wx * (v01 - v00)
                bot = v10 + wx * (v11 - v10)
                outv[b][pl.ds(o, _L)] = top + wy * (bot - top)

        def store(k, b):
            base = base0 + k * C
            pltpu.sync_copy(outv[b], out_hbm.at[pl.ds(base, C)])

        loads = [None] * n_chunks
        gathers = [None] * n_chunks
        loads[0] = start_load(0, 0)
        for k in range(n_chunks):
            b = k % 2
            for d in loads[k]:
                d.wait()
            if k + 1 < n_chunks:
                loads[k + 1] = start_load(k + 1, 1 - b)
            compute_idx(b)
            if k >= 1:
                gathers[k - 1].wait()
            gathers[k] = start_gather(b)
            if k >= 1:
                blend(1 - b)
                store(k - 1, 1 - b)
        gathers[n_chunks - 1].wait()
        blend((n_chunks - 1) % 2)
        store(n_chunks - 1, (n_chunks - 1) % 2)

    return grid_sample


def kernel(xy, grid):
    P = xy.shape[0]
    H, W = grid.shape[-2], grid.shape[-1]
    return _build(P, H, W, 2048)(xy.T, grid.reshape(-1))
